# baseline (speedup 1.0000x reference)
import functools

import jax
import jax.numpy as jnp
from jax import lax
from jax.experimental import pallas as pl
from jax.experimental.pallas import tpu as pltpu
from jax.experimental.pallas import tpu_sc as plsc

B, L = 4096, 200
EMB = 14
HID = 16
N = B * L

_info = plsc.get_sparse_core_info()
NC, NS, LANES = _info.num_cores, _info.num_subcores, _info.num_lanes
NW = NC * NS
PER_W = N // NW
BLK = 1024
NBLK = PER_W // BLK
NSTREAM = BLK // 128

_mesh = plsc.VectorSubcoreMesh(core_axis_name="c", subcore_axis_name="s")


@functools.partial(
    pl.kernel,
    mesh=_mesh,
    out_type=jax.ShapeDtypeStruct((N, HID), jnp.float32),
    scratch_types=[
        pltpu.VMEM((BLK,), jnp.int32),
        pltpu.VMEM((BLK,), jnp.float32),
        pltpu.VMEM((BLK, HID), jnp.float32),
        pltpu.SemaphoreType.DMA,
    ],
    compiler_params=pltpu.CompilerParams(
        needs_layout_passes=False,
        use_tc_tiling_on_sc=False,
    ),
)
def _sc_embed(table_h, idx_h, dur_h, out_h, idx_v, dur_v, out_v, sem):
    wid = lax.axis_index("s") * NC + lax.axis_index("c")
    lane = lax.iota(jnp.int32, LANES)
    rr_off = lane >> 1
    c_idx = (lane & 1) + EMB

    def block_body(g, carry):
        base = wid * PER_W + g * BLK
        pltpu.sync_copy(idx_h.at[pl.ds(base, BLK)], idx_v)
        pltpu.sync_copy(dur_h.at[pl.ds(base, BLK)], dur_v)

        descs = [
            pltpu.async_copy(
                table_h.at[idx_v.at[pl.ds(j * 128, 128)]],
                out_v.at[pl.ds(j * 128, 128)],
                sem,
            )
            for j in range(NSTREAM)
        ]
        for d in descs:
            d.wait()

        def fix_body(j, c):
            r_idx = j * 8 + rr_off
            val = plsc.load_gather(dur_v, [r_idx])
            plsc.store_scatter(out_v, [r_idx, c_idx], val)
            return c

        lax.fori_loop(0, BLK // 8, fix_body, 0)

        pltpu.sync_copy(out_v, out_h.at[pl.ds(base, BLK)])
        return carry

    lax.fori_loop(0, NBLK, block_body, 0)


def kernel(x, table):
    table16 = jnp.pad(table, ((0, 0), (0, HID - EMB)))
    idx = x[..., 0].astype(jnp.int32).reshape(N)
    dur = x[..., 1].reshape(N)
    out = _sc_embed(table16, idx, dur)
    return out.reshape(B, L, HID)
